# baseline (speedup 1.0000x reference)
import functools

import jax
import jax.numpy as jnp
from jax import lax
from jax.experimental import pallas as pl
from jax.experimental.pallas import tpu as pltpu
from jax.experimental.pallas import tpu_sc as plsc

BATCH_N = 16384
HIST_N = 50
DIM_N = 32
VOCAB_N = 1000000

NUM_CORES = 2
NUM_SUBCORES = 16
NUM_WORKERS = NUM_CORES * NUM_SUBCORES


def _build_detile(vocab=VOCAB_N, dim=DIM_N, unit_cols=256):
    assert dim == 32 and unit_cols % 128 == 0
    out_rows_per_unit = unit_cols * dim // 128
    n_units = vocab // unit_cols
    tail_cols = vocab - n_units * unit_cols
    assert tail_cols % 4 == 0

    mesh = plsc.VectorSubcoreMesh(
        core_axis_name="c", subcore_axis_name="s",
        num_cores=NUM_CORES, num_subcores=NUM_SUBCORES)

    @functools.partial(
        pl.kernel,
        out_type=jax.ShapeDtypeStruct((vocab * dim // 128, 128), jnp.float32),
        mesh=mesh,
        scratch_types=[
            pltpu.VMEM((2, dim, unit_cols + 1), jnp.float32),
            pltpu.VMEM((2, out_rows_per_unit, 128), jnp.float32),
            pltpu.VMEM((max(tail_cols, 4), dim), jnp.float32),
            pltpu.VMEM((max(tail_cols * dim // 128, 1), 128), jnp.float32),
            pltpu.SemaphoreType.DMA,
            pltpu.SemaphoreType.DMA,
        ],
        compiler_params=pltpu.CompilerParams(
            use_tc_tiling_on_sc=True, needs_layout_passes=False),
    )
    def _detile(tt_hbm, tail_hbm, out_hbm, buf_v, obuf_v, tbuf_v, tobuf_v,
                sem0, sem1):
        sems = (sem0, sem1)
        wid = lax.axis_index("s") * NUM_CORES + lax.axis_index("c")
        ii = lax.broadcasted_iota(jnp.int32, (16,), 0)
        ii16 = ii + 16

        def unit_id(k):
            return jnp.minimum(k * NUM_WORKERS + wid, n_units - 1)

        def read(k, b):
            base = pl.multiple_of(unit_id(k) * unit_cols, 128)
            pltpu.async_copy(
                tt_hbm.at[:, pl.ds(base, unit_cols)],
                buf_v.at[b, :, pl.ds(0, unit_cols)], sems[b])

        def wait(b):
            pltpu.make_async_copy(
                tt_hbm.at[:, pl.ds(0, unit_cols)],
                buf_v.at[b, :, pl.ds(0, unit_cols)], sems[b]).wait()

        def transpose(src, dst, cols):
            def body(g, carry):
                for cc in range(8):
                    c = g * 8 + cc
                    orow = g * 2 + cc // 4
                    ocol = (cc % 4) * 32
                    cvec = jnp.zeros((16,), jnp.int32) + c
                    dst[orow, pl.ds(ocol, 16)] = \
                        plsc.load_gather(src, [ii, cvec])
                    dst[orow, pl.ds(ocol + 16, 16)] = \
                        plsc.load_gather(src, [ii16, cvec])
                return carry
            lax.fori_loop(0, cols // 8, body, 0)

        def write(k, b):
            base = pl.multiple_of(unit_id(k) * out_rows_per_unit, 8)
            pltpu.sync_copy(
                obuf_v.at[b],
                out_hbm.at[pl.ds(base, out_rows_per_unit), :])

        n_mine = -(-n_units // NUM_WORKERS)

        n_outer = (n_mine + 1) // 2

        read(0, 0)
        read(1, 1)

        def outer(j, carry):
            for b in range(2):
                k = j * 2 + b
                wait(b)
                transpose(buf_v.at[b], obuf_v.at[b], unit_cols)
                write(k, b)
                read(k + 2, b)
            return carry

        lax.fori_loop(0, n_outer, outer, 0)
        wait(0)
        wait(1)

        if tail_cols:
            @pl.when(wid == 0)
            def _tail():
                pltpu.sync_copy(tail_hbm, tbuf_v)
                for r in range(tail_cols):
                    orow, ocol = r // 4, (r % 4) * 32
                    tobuf_v[orow, pl.ds(ocol, 16)] = tbuf_v[r, pl.ds(0, 16)]
                    tobuf_v[orow, pl.ds(ocol + 16, 16)] = \
                        tbuf_v[r, pl.ds(16, 16)]
                pltpu.sync_copy(
                    tobuf_v,
                    out_hbm.at[pl.ds(n_units * out_rows_per_unit,
                                     tail_cols * dim // 128), :])

    return _detile


def _build(batch=BATCH_N, hist=HIST_N, dim=DIM_N, rows_per_chunk=32,
           interpret=False):
    assert batch % NUM_WORKERS == 0
    b_per_w = batch // NUM_WORKERS
    assert b_per_w % rows_per_chunk == 0
    chunks = b_per_w // rows_per_chunk
    assert hist <= 128
    rows_buf = rows_per_chunk * hist

    mesh = plsc.VectorSubcoreMesh(
        core_axis_name="c", subcore_axis_name="s",
        num_cores=NUM_CORES, num_subcores=NUM_SUBCORES)

    @functools.partial(
        pl.kernel,
        out_type=jax.ShapeDtypeStruct((batch, dim), jnp.float32),
        mesh=mesh,
        scratch_types=[
            pltpu.VMEM((2, rows_per_chunk, hist), jnp.int32),
            pltpu.VMEM((2, rows_buf, dim), jnp.float32),
            pltpu.VMEM((2, rows_per_chunk, dim), jnp.float32),
            pltpu.SemaphoreType.DMA,
            pltpu.SemaphoreType.DMA,
        ],
        compiler_params=pltpu.CompilerParams(use_tc_tiling_on_sc=False),
        interpret=interpret,
    )
    def _sc_kernel(idx_hbm, table_hbm, out_hbm, idx_v, rows_v, out_v,
                   sem0, sem1):
        sems = (sem0, sem1)
        wid = lax.axis_index("s") * NUM_CORES + lax.axis_index("c")
        bbase = wid * b_per_w

        def fire(c, b):
            pltpu.sync_copy(
                idx_hbm.at[pl.ds(bbase + c * rows_per_chunk,
                                 rows_per_chunk), :],
                idx_v.at[b])
            return [
                pltpu.async_copy(
                    table_hbm.at[idx_v.at[b, j]],
                    rows_v.at[b, pl.ds(j * hist, hist)],
                    sems[b])
                for j in range(rows_per_chunk)
            ]

        def accumulate(c, b):
            def body(r, carry):
                base = r * hist
                a0 = rows_v[b, base, pl.ds(0, 16)]
                a1 = rows_v[b, base, pl.ds(16, 16)]
                for l in range(1, hist):
                    a0 = a0 + rows_v[b, base + l, pl.ds(0, 16)]
                    a1 = a1 + rows_v[b, base + l, pl.ds(16, 16)]
                out_v[b, r, pl.ds(0, 16)] = a0
                out_v[b, r, pl.ds(16, 16)] = a1
                return carry
            lax.fori_loop(0, rows_per_chunk, body, 0)
            pltpu.sync_copy(
                out_v.at[b],
                out_hbm.at[pl.ds(bbase + c * rows_per_chunk,
                                 rows_per_chunk), :])

        handles = fire(0, 0)
        for c in range(chunks):
            next_handles = fire(c + 1, (c + 1) % 2) if c + 1 < chunks else ()
            for h in handles:
                h.wait()
            accumulate(c, c % 2)
            handles = next_handles

    return _sc_kernel


_detile_impl = _build_detile()
_gather_impl = _build()


def kernel(inputs, embeddings):
    tail = embeddings[VOCAB_N - (VOCAB_N % 256):, :]
    table_lin = _detile_impl(embeddings.T, tail)
    table = table_lin.reshape(VOCAB_N, DIM_N)
    return _gather_impl(inputs.astype(jnp.int32), table)


# speedup vs baseline: 1.5683x; 1.5683x over previous
import functools

import jax
import jax.numpy as jnp
from jax import lax
from jax.experimental import pallas as pl
from jax.experimental.pallas import tpu as pltpu
from jax.experimental.pallas import tpu_sc as plsc

BATCH_N = 16384
HIST_N = 50
DIM_N = 32
VOCAB_N = 1000000

NUM_CORES = 2
NUM_SUBCORES = 16
NUM_WORKERS = NUM_CORES * NUM_SUBCORES


def _build(batch=BATCH_N, hist=HIST_N, dim=DIM_N, rows_per_chunk=32,
           interpret=False):
    assert batch % NUM_WORKERS == 0
    b_per_w = batch // NUM_WORKERS
    assert b_per_w % rows_per_chunk == 0
    chunks = b_per_w // rows_per_chunk
    assert hist <= 128
    rows_buf = rows_per_chunk * hist

    mesh = plsc.VectorSubcoreMesh(
        core_axis_name="c", subcore_axis_name="s",
        num_cores=NUM_CORES, num_subcores=NUM_SUBCORES)

    @functools.partial(
        pl.kernel,
        out_type=jax.ShapeDtypeStruct((batch, dim), jnp.float32),
        mesh=mesh,
        scratch_types=[
            pltpu.VMEM((2, rows_per_chunk, hist), jnp.int32),
            pltpu.VMEM((2, rows_buf, dim), jnp.float32),
            pltpu.VMEM((2, rows_per_chunk, dim), jnp.float32),
            pltpu.SemaphoreType.DMA,
            pltpu.SemaphoreType.DMA,
        ],
        compiler_params=pltpu.CompilerParams(use_tc_tiling_on_sc=False),
        interpret=interpret,
    )
    def _sc_kernel(idx_hbm, table_hbm, out_hbm, idx_v, rows_v, out_v,
                   sem0, sem1):
        sems = (sem0, sem1)
        wid = lax.axis_index("s") * NUM_CORES + lax.axis_index("c")
        bbase = wid * b_per_w

        def fire(c, b):
            pltpu.sync_copy(
                idx_hbm.at[pl.ds(bbase + c * rows_per_chunk,
                                 rows_per_chunk), :],
                idx_v.at[b])
            return [
                pltpu.async_copy(
                    table_hbm.at[idx_v.at[b, j]],
                    rows_v.at[b, pl.ds(j * hist, hist)],
                    sems[b])
                for j in range(rows_per_chunk)
            ]

        def accumulate(c, b):
            def body(r, carry):
                base = r * hist
                a0 = rows_v[b, base, pl.ds(0, 16)]
                a1 = rows_v[b, base, pl.ds(16, 16)]
                for l in range(1, hist):
                    a0 = a0 + rows_v[b, base + l, pl.ds(0, 16)]
                    a1 = a1 + rows_v[b, base + l, pl.ds(16, 16)]
                out_v[b, r, pl.ds(0, 16)] = a0
                out_v[b, r, pl.ds(16, 16)] = a1
                return carry
            lax.fori_loop(0, rows_per_chunk, body, 0)
            pltpu.sync_copy(
                out_v.at[b],
                out_hbm.at[pl.ds(bbase + c * rows_per_chunk,
                                 rows_per_chunk), :])

        handles = fire(0, 0)
        for c in range(chunks):
            next_handles = fire(c + 1, (c + 1) % 2) if c + 1 < chunks else ()
            for h in handles:
                h.wait()
            accumulate(c, c % 2)
            handles = next_handles

    return _sc_kernel


_gather_impl = _build()


def kernel(inputs, embeddings):
    return _gather_impl(inputs.astype(jnp.int32), embeddings)
